# baseline (speedup 1.0000x reference)
import functools

import jax
import jax.numpy as jnp
from jax.experimental import pallas as pl
from jax.experimental.pallas import tpu as pltpu


def _gcn_block(scal_ref, adj_ref, x_ref, h0_ref, w_ref, out_ref):
    s = scal_ref[0, 0]
    theta = scal_ref[0, 1]
    a = adj_ref[...].astype(jnp.bfloat16)
    x = x_ref[...].astype(jnp.bfloat16)
    hi = jax.lax.dot_general(
        a, x, (((1,), (0,)), ((), ())), preferred_element_type=jnp.float32
    )
    support = (1.0 - s) * hi + s * h0_ref[...]
    sw = jax.lax.dot_general(
        support.astype(jnp.bfloat16),
        w_ref[...].astype(jnp.bfloat16),
        (((1,), (0,)), ((), ())),
        preferred_element_type=jnp.float32,
    )
    out_ref[...] = theta * sw + (1.0 - theta) * support


@functools.partial(jax.jit, static_argnames=("block_m",))
def _gcn(input, adj, h0, weight, s, theta, block_m=400):
    n, d_in = input.shape
    d_out = weight.shape[1]
    scal = jnp.reshape(
        jnp.stack([s, theta]).astype(jnp.float32), (1, 2)
    )
    return pl.pallas_call(
        _gcn_block,
        grid=(n // block_m,),
        in_specs=[
            pl.BlockSpec(memory_space=pltpu.SMEM),
            pl.BlockSpec((block_m, n), lambda i: (i, 0)),
            pl.BlockSpec((n, d_in), lambda i: (0, 0)),
            pl.BlockSpec((block_m, d_in), lambda i: (i, 0)),
            pl.BlockSpec((d_in, d_out), lambda i: (0, 0)),
        ],
        out_specs=pl.BlockSpec((block_m, d_out), lambda i: (i, 0)),
        out_shape=jax.ShapeDtypeStruct((n, d_out), jnp.float32),
        compiler_params=pltpu.CompilerParams(
            dimension_semantics=("arbitrary",),
        ),
    )(scal, adj, input, h0, weight)


def kernel(input, adj, h0, weight, lamda, s, l):
    theta = (lamda / l).astype(jnp.float32)
    s = jnp.asarray(s, jnp.float32)
    return _gcn(input, adj, h0, weight, s, theta, block_m=200)


# speedup vs baseline: 1.0120x; 1.0120x over previous
import functools

import jax
import jax.numpy as jnp
from jax.experimental import pallas as pl
from jax.experimental.pallas import tpu as pltpu


def _gcn_block(scal_ref, adj_ref, x_ref, h0_ref, w_ref, out_ref):
    s = scal_ref[0, 0]
    theta = scal_ref[0, 1]
    a = adj_ref[...].astype(jnp.bfloat16)
    x = x_ref[...].astype(jnp.bfloat16)
    hi = jax.lax.dot_general(
        a, x, (((1,), (0,)), ((), ())), preferred_element_type=jnp.float32
    )
    support = (1.0 - s) * hi + s * h0_ref[...]
    sw = jax.lax.dot_general(
        support.astype(jnp.bfloat16),
        w_ref[...].astype(jnp.bfloat16),
        (((1,), (0,)), ((), ())),
        preferred_element_type=jnp.float32,
    )
    out_ref[...] = theta * sw + (1.0 - theta) * support


@functools.partial(jax.jit, static_argnames=("block_m",))
def _gcn(input, adj, h0, weight, s, theta, block_m=400):
    n, d_in = input.shape
    d_out = weight.shape[1]
    scal = jnp.reshape(
        jnp.stack([s, theta]).astype(jnp.float32), (1, 2)
    )
    return pl.pallas_call(
        _gcn_block,
        grid=(n // block_m,),
        in_specs=[
            pl.BlockSpec(memory_space=pltpu.SMEM),
            pl.BlockSpec((block_m, n), lambda i: (i, 0)),
            pl.BlockSpec((n, d_in), lambda i: (0, 0)),
            pl.BlockSpec((block_m, d_in), lambda i: (i, 0)),
            pl.BlockSpec((d_in, d_out), lambda i: (0, 0)),
        ],
        out_specs=pl.BlockSpec((block_m, d_out), lambda i: (i, 0)),
        out_shape=jax.ShapeDtypeStruct((n, d_out), jnp.float32),
        compiler_params=pltpu.CompilerParams(
            dimension_semantics=("arbitrary",),
        ),
    )(scal, adj, input, h0, weight)


def kernel(input, adj, h0, weight, lamda, s, l):
    theta = (lamda / l).astype(jnp.float32)
    s = jnp.asarray(s, jnp.float32)
    return _gcn(input, adj, h0, weight, s, theta, block_m=400)


# speedup vs baseline: 1.0130x; 1.0010x over previous
import functools

import jax
import jax.numpy as jnp
from jax.experimental import pallas as pl
from jax.experimental.pallas import tpu as pltpu


def _gcn_block(scal_ref, adj_ref, x_ref, h0_ref, w_ref, out_ref):
    s = scal_ref[0, 0]
    theta = scal_ref[0, 1]
    a = adj_ref[...].astype(jnp.bfloat16)
    x = x_ref[...].astype(jnp.bfloat16)
    hi = jax.lax.dot_general(
        a, x, (((1,), (0,)), ((), ())), preferred_element_type=jnp.float32
    )
    support = (1.0 - s) * hi + s * h0_ref[...]
    sw = jax.lax.dot_general(
        support.astype(jnp.bfloat16),
        w_ref[...].astype(jnp.bfloat16),
        (((1,), (0,)), ((), ())),
        preferred_element_type=jnp.float32,
    )
    out_ref[...] = theta * sw + (1.0 - theta) * support


@functools.partial(jax.jit, static_argnames=("block_m",))
def _gcn(input, adj, h0, weight, s, theta, block_m=400):
    n, d_in = input.shape
    d_out = weight.shape[1]
    scal = jnp.reshape(
        jnp.stack([s, theta]).astype(jnp.float32), (1, 2)
    )
    return pl.pallas_call(
        _gcn_block,
        grid=(n // block_m,),
        in_specs=[
            pl.BlockSpec(memory_space=pltpu.SMEM),
            pl.BlockSpec((block_m, n), lambda i: (i, 0)),
            pl.BlockSpec((n, d_in), lambda i: (0, 0)),
            pl.BlockSpec((block_m, d_in), lambda i: (i, 0)),
            pl.BlockSpec((d_in, d_out), lambda i: (0, 0)),
        ],
        out_specs=pl.BlockSpec((block_m, d_out), lambda i: (i, 0)),
        out_shape=jax.ShapeDtypeStruct((n, d_out), jnp.float32),
        compiler_params=pltpu.CompilerParams(
            dimension_semantics=("parallel",),
        ),
    )(scal, adj, input, h0, weight)


def kernel(input, adj, h0, weight, lamda, s, l):
    theta = (lamda / l).astype(jnp.float32)
    s = jnp.asarray(s, jnp.float32)
    return _gcn(input, adj, h0, weight, s, theta, block_m=400)


# speedup vs baseline: 1.0297x; 1.0165x over previous
import functools

import jax
import jax.numpy as jnp
from jax.experimental import pallas as pl
from jax.experimental.pallas import tpu as pltpu


def _gcn_block(scal_ref, adj_a_ref, adj_b_ref, x_ref, h0_ref, w_ref, out_ref):
    s = scal_ref[0, 0]
    theta = scal_ref[0, 1]
    x = x_ref[...].astype(jnp.bfloat16)
    w = w_ref[...].astype(jnp.bfloat16)
    half = adj_a_ref.shape[0]
    for idx, a_ref in enumerate((adj_a_ref, adj_b_ref)):
        a = a_ref[...].astype(jnp.bfloat16)
        hi = jax.lax.dot_general(
            a, x, (((1,), (0,)), ((), ())), preferred_element_type=jnp.float32
        )
        rows = pl.ds(idx * half, half)
        support = (1.0 - s) * hi + s * h0_ref[rows, :]
        sw = jax.lax.dot_general(
            support.astype(jnp.bfloat16),
            w,
            (((1,), (0,)), ((), ())),
            preferred_element_type=jnp.float32,
        )
        out_ref[rows, :] = theta * sw + (1.0 - theta) * support


@functools.partial(jax.jit, static_argnames=("block_m",))
def _gcn(input, adj, h0, weight, s, theta, block_m=400):
    n, d_in = input.shape
    d_out = weight.shape[1]
    half = block_m // 2
    scal = jnp.reshape(jnp.stack([s, theta]).astype(jnp.float32), (1, 2))
    return pl.pallas_call(
        _gcn_block,
        grid=(n // block_m,),
        in_specs=[
            pl.BlockSpec(memory_space=pltpu.SMEM),
            pl.BlockSpec((half, n), lambda i: (2 * i, 0)),
            pl.BlockSpec((half, n), lambda i: (2 * i + 1, 0)),
            pl.BlockSpec((n, d_in), lambda i: (0, 0)),
            pl.BlockSpec((block_m, d_in), lambda i: (i, 0)),
            pl.BlockSpec((d_in, d_out), lambda i: (0, 0)),
        ],
        out_specs=pl.BlockSpec((block_m, d_out), lambda i: (i, 0)),
        out_shape=jax.ShapeDtypeStruct((n, d_out), jnp.float32),
        compiler_params=pltpu.CompilerParams(
            dimension_semantics=("parallel",),
        ),
    )(scal, adj, adj, input, h0, weight)


def kernel(input, adj, h0, weight, lamda, s, l):
    theta = (lamda / l).astype(jnp.float32)
    s = jnp.asarray(s, jnp.float32)
    return _gcn(input, adj, h0, weight, s, theta)
